# baseline (speedup 1.0000x reference)
import jax
import jax.numpy as jnp
from jax.experimental import pallas as pl
from jax.experimental.pallas import tpu as pltpu

_B, _S, _D_MODEL, _D_SPACE = 4, 2048, 4096, 64
_N_POOL = 512
_RV_END = _N_POOL * 6
_TOPKS = (8, 8, 3, 8, 8, 3)
_TILE = 256
_STRIP = 64


def _norm_kernel(emb_ref, out_ref):
    emb = emb_ref[...]
    inv = 1.0 / jnp.maximum(
        jnp.sqrt(jnp.sum(emb * emb, axis=1, keepdims=True)), 1e-12
    )
    out_ref[...] = emb * inv


def _router_kernel(x_ref, w_ref, b_ref, emb_ref, out_ref):
    x = x_ref[...]
    w = w_ref[...]
    proj = jax.lax.dot_general(
        x, w, (((1,), (0,)), ((), ())), preferred_element_type=jnp.float32
    )
    proj = proj + b_ref[...]
    emb_n = emb_ref[...]
    neg = jnp.float32(-jnp.inf)
    for g in range(6):
        e = emb_n[g * _N_POOL:(g + 1) * _N_POOL, :]
        for s in range(0, _TILE, _STRIP):
            h = proj[s:s + _STRIP, g * _D_SPACE:(g + 1) * _D_SPACE]
            logits = jax.lax.dot_general(
                h, e, (((1,), (1,)), ((), ())), preferred_element_type=jnp.float32
            )
            t = jnp.max(logits, axis=1, keepdims=True)
            m = t
            vals = jnp.where(logits >= t, neg, logits)
            for _ in range(_TOPKS[g] - 1):
                t = jnp.max(vals, axis=1, keepdims=True)
                vals = jnp.where(vals >= t, neg, vals)
            ex = jnp.exp(logits - m)
            rz = 1.0 / jnp.sum(ex, axis=1, keepdims=True)
            out_ref[s:s + _STRIP, g * _N_POOL:(g + 1) * _N_POOL] = jnp.where(
                logits >= t, ex * rz, 0.0
            )


def kernel(x, importance, W_proj, b_proj, neuron_emb):
    del importance
    xf = x.reshape(_B * _S, _D_MODEL)
    emb = neuron_emb[:_RV_END]
    b2 = b_proj.reshape(1, _D_SPACE * 6)
    emb_n = pl.pallas_call(
        _norm_kernel,
        out_shape=jax.ShapeDtypeStruct((_RV_END, _D_SPACE), jnp.float32),
    )(emb)
    out = pl.pallas_call(
        _router_kernel,
        grid=(_B * _S // _TILE,),
        in_specs=[
            pl.BlockSpec((_TILE, _D_MODEL), lambda i: (i, 0)),
            pl.BlockSpec((_D_MODEL, _D_SPACE * 6), lambda i: (0, 0)),
            pl.BlockSpec((1, _D_SPACE * 6), lambda i: (0, 0)),
            pl.BlockSpec((_RV_END, _D_SPACE), lambda i: (0, 0)),
        ],
        out_specs=pl.BlockSpec((_TILE, _RV_END), lambda i: (i, 0)),
        out_shape=jax.ShapeDtypeStruct((_B * _S, _RV_END), jnp.float32),
        compiler_params=pltpu.CompilerParams(dimension_semantics=("parallel",)),
    )(xf, W_proj, b2, emb_n)
    return out.reshape(_B, _S, _RV_END)
